# Initial kernel scaffold; baseline (speedup 1.0000x reference)
import jax
import jax.numpy as jnp
from jax.experimental import pallas as pl


def kernel(x, pe):
    raise NotImplementedError("write your pallas kernel here")


# speedup vs baseline: 1.6713x; 1.6713x over previous
import jax
import jax.numpy as jnp
from jax.experimental import pallas as pl


def _add_kernel(x_ref, pe_ref, o_ref):
    o_ref[...] = x_ref[...] + pe_ref[...]


def kernel(x, pe):
    B, S, D = x.shape
    SB = 512
    grid = (S // SB, B)
    return pl.pallas_call(
        _add_kernel,
        grid=grid,
        in_specs=[
            pl.BlockSpec((1, SB, D), lambda j, i: (i, j, 0)),
            pl.BlockSpec((SB, D), lambda j, i: (j, 0)),
        ],
        out_specs=pl.BlockSpec((1, SB, D), lambda j, i: (i, j, 0)),
        out_shape=jax.ShapeDtypeStruct((B, S, D), x.dtype),
    )(x, pe[:S])


# speedup vs baseline: 1.8484x; 1.1059x over previous
import jax
import jax.numpy as jnp
from jax.experimental import pallas as pl


def _add_kernel(x_ref, pe_ref, o_ref):
    o_ref[...] = x_ref[...] + pe_ref[...]


def kernel(x, pe):
    B, S, D = x.shape
    SB = 1024
    grid = (S // SB, B)
    return pl.pallas_call(
        _add_kernel,
        grid=grid,
        in_specs=[
            pl.BlockSpec((1, SB, D), lambda j, i: (i, j, 0)),
            pl.BlockSpec((SB, D), lambda j, i: (j, 0)),
        ],
        out_specs=pl.BlockSpec((1, SB, D), lambda j, i: (i, j, 0)),
        out_shape=jax.ShapeDtypeStruct((B, S, D), x.dtype),
    )(x, pe[:S])


# speedup vs baseline: 1.9658x; 1.0635x over previous
import jax
import jax.numpy as jnp
from jax.experimental import pallas as pl


def _add_kernel(x_ref, pe_ref, o_ref):
    o_ref[...] = x_ref[...] + pe_ref[...]


def kernel(x, pe):
    B, S, D = x.shape
    SB = 2048
    grid = (S // SB, B)
    return pl.pallas_call(
        _add_kernel,
        grid=grid,
        in_specs=[
            pl.BlockSpec((1, SB, D), lambda j, i: (i, j, 0)),
            pl.BlockSpec((SB, D), lambda j, i: (j, 0)),
        ],
        out_specs=pl.BlockSpec((1, SB, D), lambda j, i: (i, j, 0)),
        out_shape=jax.ShapeDtypeStruct((B, S, D), x.dtype),
    )(x, pe[:S])
